# Initial kernel scaffold; baseline (speedup 1.0000x reference)
import jax
import jax.numpy as jnp
from jax.experimental import pallas as pl


def kernel(move_tokens, token_table, pos_table):
    raise NotImplementedError("write your pallas kernel here")


# speedup vs baseline: 1.2000x; 1.2000x over previous
import functools

import jax
import jax.numpy as jnp
from jax import lax
from jax.experimental import pallas as pl
from jax.experimental.pallas import tpu as pltpu
from jax.experimental.pallas import tpu_sc as plsc


def _build_sc_call(B, T, V, D, S, CHUNK):
    info = plsc.get_sparse_core_info()
    NC, NS, L = info.num_cores, info.num_subcores, info.num_lanes
    NW = NC * NS
    BT = B * T
    assert BT % NW == 0
    rows_per_w = BT // NW
    assert rows_per_w % CHUNK == 0 and CHUNK % T == 0 and CHUNK <= 128
    n_chunks = rows_per_w // CHUNK
    lanes_per_row = D // L

    mesh = plsc.VectorSubcoreMesh(core_axis_name="c", subcore_axis_name="s")

    @functools.partial(
        pl.kernel,
        mesh=mesh,
        out_type=jax.ShapeDtypeStruct((BT, D), jnp.float32),
        scratch_types=[
            pltpu.VMEM((CHUNK,), jnp.int32),
            pltpu.VMEM((CHUNK, D), jnp.float32),
            pltpu.VMEM((24, D), jnp.float32),
            pltpu.SemaphoreType.DMA,
        ],
    )
    def sc_kernel(table_hbm, idx_hbm, pos_hbm, out_hbm, idx_v, rows_v, pos_v, gsem):
        wid = lax.axis_index("s") * NC + lax.axis_index("c")
        wbase = wid * rows_per_w

        pltpu.sync_copy(pos_hbm.at[pl.ds(0, 24)], pos_v)

        def chunk_body(i, carry):
            base = wbase + i * CHUNK
            pltpu.sync_copy(idx_hbm.at[pl.ds(base, CHUNK)], idx_v)
            pltpu.async_copy(table_hbm.at[idx_v], rows_v, gsem).wait()

            def add_body(r, carry2):
                pr = lax.rem(r, T)
                for c in range(lanes_per_row):
                    sl = pl.ds(c * L, L)
                    rows_v[r, sl] = rows_v[r, sl] + pos_v[pr, sl]
                return carry2

            lax.fori_loop(0, CHUNK, add_body, 0, unroll=False)
            pltpu.sync_copy(rows_v, out_hbm.at[pl.ds(base, CHUNK)])
            return carry

        lax.fori_loop(0, n_chunks, chunk_body, 0, unroll=False)

    return sc_kernel


def kernel(move_tokens, token_table, pos_table):
    B, T = move_tokens.shape
    V, D = token_table.shape
    S = pos_table.shape[0]
    flat_idx = move_tokens.reshape(BT_ := B * T).astype(jnp.int32)
    sc_call = _build_sc_call(B, T, V, D, S, CHUNK=80)
    out = sc_call(token_table, flat_idx, pos_table)
    return out.reshape(B, T, D)


# speedup vs baseline: 2.3469x; 1.9558x over previous
import functools

import jax
import jax.numpy as jnp
from jax import lax
from jax.experimental import pallas as pl
from jax.experimental.pallas import tpu as pltpu
from jax.experimental.pallas import tpu_sc as plsc


def _build_sc_call(B, T, V, D, CHUNK):
    info = plsc.get_sparse_core_info()
    NC, NS, L = info.num_cores, info.num_subcores, info.num_lanes
    NW = NC * NS
    BT = B * T
    assert BT % NW == 0
    rows_per_w = BT // NW
    assert rows_per_w % CHUNK == 0 and CHUNK % T == 0 and CHUNK <= 128
    n_chunks = rows_per_w // CHUNK
    assert n_chunks % 2 == 0 and n_chunks >= 4
    lanes = D // L
    reps = CHUNK // T
    POS_PAD = ((T + 7) // 8) * 8

    mesh = plsc.VectorSubcoreMesh(core_axis_name="c", subcore_axis_name="s")

    @functools.partial(
        pl.kernel,
        mesh=mesh,
        out_type=jax.ShapeDtypeStruct((BT, D), jnp.float32),
        scratch_types=[
            pltpu.VMEM((rows_per_w,), jnp.int32),
            pltpu.VMEM((CHUNK, D), jnp.float32),
            pltpu.VMEM((CHUNK, D), jnp.float32),
            pltpu.VMEM((CHUNK, D), jnp.float32),
            pltpu.VMEM((CHUNK, D), jnp.float32),
            pltpu.VMEM((POS_PAD, D), jnp.float32),
            pltpu.SemaphoreType.DMA,
            pltpu.SemaphoreType.DMA,
            pltpu.SemaphoreType.DMA,
            pltpu.SemaphoreType.DMA,
        ],
    )
    def sc_kernel(table_hbm, idx_hbm, pos_hbm, out_hbm,
                  idx_v, gb0, gb1, ob0, ob1, pos_v, gs0, gs1, os0, os1):
        wid = lax.axis_index("s") * NC + lax.axis_index("c")
        wbase = wid * rows_per_w
        gb, ob, gs, osem = [gb0, gb1], [ob0, ob1], [gs0, gs1], [os0, os1]

        pltpu.sync_copy(pos_hbm.at[pl.ds(0, POS_PAD)], pos_v)
        pltpu.sync_copy(idx_hbm.at[pl.ds(wbase, rows_per_w)], idx_v)

        def g_src(i):
            return table_hbm.at[idx_v.at[pl.ds(pl.multiple_of(i * CHUNK, CHUNK), CHUNK)]]

        def o_dst(i):
            return out_hbm.at[pl.ds(pl.multiple_of(wbase + i * CHUNK, CHUNK), CHUNK)]

        def add_chunk(s):
            def add_rows(r20, carry):
                pv = [pos_v[r20, pl.ds(c * L, L)] for c in range(lanes)]
                for rep in range(reps):
                    r = r20 + rep * T
                    for c in range(lanes):
                        sl = pl.ds(c * L, L)
                        ob[s][r, sl] = gb[s][r, sl] + pv[c]
                return carry

            lax.fori_loop(0, T, add_rows, 0, unroll=False)

        def step(i, s, prefetch):
            pltpu.make_async_copy(g_src(i), gb[s], gs[s]).wait()

            @pl.when(i >= 2)
            def _():
                pltpu.make_async_copy(ob[s], o_dst(i - 2), osem[s]).wait()

            add_chunk(s)
            if prefetch:
                pltpu.async_copy(g_src(i + 2), gb[s], gs[s])
            pltpu.async_copy(ob[s], o_dst(i), osem[s])

        for s in range(2):
            pltpu.async_copy(g_src(s), gb[s], gs[s])

        def outer(o, carry):
            for s in range(2):
                step(o * 2 + s, s, prefetch=True)
            return carry

        lax.fori_loop(0, n_chunks // 2 - 1, outer, 0, unroll=False)
        for s in range(2):
            step(n_chunks - 2 + s, s, prefetch=False)
        for s in range(2):
            pltpu.make_async_copy(ob[s], o_dst(n_chunks - 2 + s), osem[s]).wait()

    return sc_kernel


def kernel(move_tokens, token_table, pos_table):
    B, T = move_tokens.shape
    V, D = token_table.shape
    flat_idx = move_tokens.reshape(B * T).astype(jnp.int32)
    sc_call = _build_sc_call(B, T, V, D, CHUNK=80)
    out = sc_call(token_table, flat_idx, pos_table)
    return out.reshape(B, T, D)


# speedup vs baseline: 2.6402x; 1.1250x over previous
import functools

import jax
import jax.numpy as jnp
from jax import lax
from jax.experimental import pallas as pl
from jax.experimental.pallas import tpu as pltpu
from jax.experimental.pallas import tpu_sc as plsc


def _build_sc_call(B, T, V, D, NB):
    info = plsc.get_sparse_core_info()
    NC, NS, L = info.num_cores, info.num_subcores, info.num_lanes
    NW = NC * NS
    assert B % NW == 0
    b_per_w = B // NW
    assert b_per_w % NB == 0
    CHUNK = NB * T
    assert CHUNK <= 128
    n_chunks = b_per_w // NB
    assert n_chunks % 2 == 0 and n_chunks >= 4
    lanes = D // L
    POS_PAD = ((T + 7) // 8) * 8

    mesh = plsc.VectorSubcoreMesh(core_axis_name="c", subcore_axis_name="s")

    @functools.partial(
        pl.kernel,
        mesh=mesh,
        compiler_params=pltpu.CompilerParams(use_tc_tiling_on_sc=True),
        out_type=jax.ShapeDtypeStruct((B, T, D), jnp.float32),
        scratch_types=[
            pltpu.VMEM((b_per_w * T,), jnp.int32),
            pltpu.VMEM((CHUNK, D), jnp.float32),
            pltpu.VMEM((CHUNK, D), jnp.float32),
            pltpu.VMEM((NB, T, D), jnp.float32),
            pltpu.VMEM((NB, T, D), jnp.float32),
            pltpu.VMEM((POS_PAD, D), jnp.float32),
            pltpu.SemaphoreType.DMA,
            pltpu.SemaphoreType.DMA,
            pltpu.SemaphoreType.DMA,
            pltpu.SemaphoreType.DMA,
        ],
    )
    def sc_kernel(table_hbm, idx_hbm, pos_hbm, out_hbm,
                  idx_v, gb0, gb1, ob0, ob1, pos_v, gs0, gs1, os0, os1):
        wid = lax.axis_index("s") * NC + lax.axis_index("c")
        wrow = wid * b_per_w * T
        wb = wid * b_per_w
        gb, ob, gs, osem = [gb0, gb1], [ob0, ob1], [gs0, gs1], [os0, os1]

        pltpu.sync_copy(pos_hbm.at[pl.ds(0, POS_PAD)], pos_v)
        pltpu.sync_copy(idx_hbm.at[pl.ds(wrow, b_per_w * T)], idx_v)

        def g_src(i):
            return table_hbm.at[idx_v.at[pl.ds(pl.multiple_of(i * CHUNK, CHUNK), CHUNK)]]

        def o_dst(i):
            return out_hbm.at[pl.ds(pl.multiple_of(wb + i * NB, NB), NB)]

        def add_chunk(s):
            def add_rows(t, carry):
                pv = [pos_v[t, pl.ds(c * L, L)] for c in range(lanes)]
                for nb in range(NB):
                    r = t + nb * T
                    for c in range(lanes):
                        sl = pl.ds(c * L, L)
                        ob[s][nb, t, sl] = gb[s][r, sl] + pv[c]
                return carry

            lax.fori_loop(0, T, add_rows, 0, unroll=False)

        def step(i, s, prefetch):
            pltpu.make_async_copy(g_src(i), gb[s], gs[s]).wait()

            @pl.when(i >= 2)
            def _():
                pltpu.make_async_copy(ob[s], o_dst(i - 2), osem[s]).wait()

            add_chunk(s)
            if prefetch:
                pltpu.async_copy(g_src(i + 2), gb[s], gs[s])
            pltpu.async_copy(ob[s], o_dst(i), osem[s])

        for s in range(2):
            pltpu.async_copy(g_src(s), gb[s], gs[s])

        def outer(o, carry):
            for s in range(2):
                step(o * 2 + s, s, prefetch=True)
            return carry

        lax.fori_loop(0, n_chunks // 2 - 1, outer, 0, unroll=False)
        for s in range(2):
            step(n_chunks - 2 + s, s, prefetch=False)
        for s in range(2):
            pltpu.make_async_copy(ob[s], o_dst(n_chunks - 2 + s), osem[s]).wait()

    return sc_kernel


def kernel(move_tokens, token_table, pos_table):
    B, T = move_tokens.shape
    V, D = token_table.shape
    flat_idx = move_tokens.reshape(B * T).astype(jnp.int32)
    sc_call = _build_sc_call(B, T, V, D, NB=4)
    return sc_call(token_table, flat_idx, pos_table)
